# speedup vs baseline: 1.1898x; 1.1898x over previous
import jax
import jax.numpy as jnp
from jax.experimental import pallas as pl


def _body(idx_ref, cont_ref, disc_tab_ref, cont_tab_ref, out_ref):
    idx = idx_ref[...]
    r = idx.shape[0]
    vocab = disc_tab_ref.shape[0]
    iota = jax.lax.broadcasted_iota(jnp.int32, (r, 4, vocab), 2)
    one_hot = (idx[:, :, None] == iota).astype(jnp.float32)
    disc = jax.lax.dot_general(
        one_hot, disc_tab_ref[...],
        dimension_numbers=(((2,), (0,)), ((), ())),
        preferred_element_type=jnp.float32,
    )
    cont = cont_ref[...][:, :, None] * cont_tab_ref[...][None, :, :]
    out_ref[...] = jnp.concatenate([disc, cont], axis=1)


def kernel(discrete_actions, continuous_actions, disc_table, cont_table, offsets):
    b, s, n_disc = discrete_actions.shape
    n_cont = continuous_actions.shape[-1]
    dim = disc_table.shape[-1]
    n = b * s
    flat_idx = (discrete_actions + offsets[None, None, :]).reshape(n, n_disc)
    cont = continuous_actions.reshape(n, n_cont)

    R = 64
    grid = (n // R,)
    out = pl.pallas_call(
        _body,
        grid=grid,
        in_specs=[
            pl.BlockSpec((R, n_disc), lambda i: (i, 0)),
            pl.BlockSpec((R, n_cont), lambda i: (i, 0)),
            pl.BlockSpec(disc_table.shape, lambda i: (0, 0)),
            pl.BlockSpec(cont_table.shape, lambda i: (0, 0)),
        ],
        out_specs=pl.BlockSpec((R, n_disc + n_cont, dim), lambda i: (i, 0, 0)),
        out_shape=jax.ShapeDtypeStruct((n, n_disc + n_cont, dim), jnp.float32),
    )(flat_idx, cont, disc_table, cont_table)
    return out.reshape(b, s, n_disc + n_cont, dim)


# speedup vs baseline: 1.1927x; 1.0025x over previous
import jax
import jax.numpy as jnp
from jax.experimental import pallas as pl


def _body(idx_ref, cont_ref, disc_tab_ref, cont_tab_ref, out_ref):
    idx = idx_ref[...]
    r = idx.shape[0]
    vocab = disc_tab_ref.shape[0]
    iota = jax.lax.broadcasted_iota(jnp.int32, (r, 4, vocab), 2)
    one_hot = (idx[:, :, None] == iota).astype(jnp.float32)
    disc = jax.lax.dot_general(
        one_hot, disc_tab_ref[...],
        dimension_numbers=(((2,), (0,)), ((), ())),
        preferred_element_type=jnp.float32,
    )
    cont = cont_ref[...][:, :, None] * cont_tab_ref[...][None, :, :]
    out_ref[...] = jnp.concatenate([disc, cont], axis=1)


def kernel(discrete_actions, continuous_actions, disc_table, cont_table, offsets):
    b, s, n_disc = discrete_actions.shape
    n_cont = continuous_actions.shape[-1]
    dim = disc_table.shape[-1]
    n = b * s
    flat_idx = (discrete_actions + offsets[None, None, :]).reshape(n, n_disc)
    cont = continuous_actions.reshape(n, n_cont)

    R = 128
    grid = (n // R,)
    out = pl.pallas_call(
        _body,
        grid=grid,
        in_specs=[
            pl.BlockSpec((R, n_disc), lambda i: (i, 0)),
            pl.BlockSpec((R, n_cont), lambda i: (i, 0)),
            pl.BlockSpec(disc_table.shape, lambda i: (0, 0)),
            pl.BlockSpec(cont_table.shape, lambda i: (0, 0)),
        ],
        out_specs=pl.BlockSpec((R, n_disc + n_cont, dim), lambda i: (i, 0, 0)),
        out_shape=jax.ShapeDtypeStruct((n, n_disc + n_cont, dim), jnp.float32),
    )(flat_idx, cont, disc_table, cont_table)
    return out.reshape(b, s, n_disc + n_cont, dim)
